# initial kernel scaffold (unmeasured)
import jax
import jax.numpy as jnp
from jax import lax
from jax.experimental import pallas as pl
from jax.experimental.pallas import tpu as pltpu

N_DEV = 16
E_LOC = 8


def kernel(x, router_W, route_idx, expert_W):
    T, D = x.shape
    E = router_W.shape[1]
    H = expert_W.shape[2]

    def body(x_ref, rw_ref, idx_ref, ew_ref, out_ref,
             x_com, idx_com, out_com,
             x_send, x_recv, i_send, i_recv, o_send, o_recv,
             credit):
        me = lax.axis_index("i")
        left = lax.rem(me - 1 + N_DEV, N_DEV)
        right = lax.rem(me + 1, N_DEV)

        bar = pltpu.get_barrier_semaphore()
        pl.semaphore_signal(bar, inc=1, device_id=(left,),
                            device_id_type=pl.DeviceIdType.MESH)
        pl.semaphore_signal(bar, inc=1, device_id=(right,),
                            device_id_type=pl.DeviceIdType.MESH)
        pl.semaphore_wait(bar, 2)

        def contribution(xv, idxv, acc):
            scores = jnp.dot(xv, rw_ref[...],
                             preferred_element_type=jnp.float32)
            idx0 = idxv[:, 0:1]
            idx1 = idxv[:, 1:2]
            eids = lax.broadcasted_iota(jnp.int32, (T, E), 1)
            s0 = jnp.sum(jnp.where(eids == idx0, scores, 0.0),
                         axis=1, keepdims=True)
            s1 = jnp.sum(jnp.where(eids == idx1, scores, 0.0),
                         axis=1, keepdims=True)
            m = jnp.maximum(s0, s1)
            e0 = jnp.exp(s0 - m)
            e1 = jnp.exp(s1 - m)
            w0 = e0 / (e0 + e1)
            w1 = 1.0 - w0
            for el in range(E_LOC):
                eg = me * E_LOC + el
                g = (w0 * (idx0 == eg).astype(jnp.float32)
                     + w1 * (idx1 == eg).astype(jnp.float32))
                y = jnp.dot(xv, ew_ref[el],
                            preferred_element_type=jnp.float32)
                acc = acc + g * y
            return acc

        x_com[0] = x_ref[...]
        idx_com[0] = idx_ref[...]
        out_com[0] = contribution(x_ref[...], idx_ref[...],
                                  jnp.zeros((T, H), jnp.float32))

        def step(t, carry):
            cur = lax.rem(t, 2)
            nxt = lax.rem(t + 1, 2)

            @pl.when(t >= 1)
            def _():
                pl.semaphore_wait(credit.at[nxt], 1)

            o_rdma = pltpu.make_async_remote_copy(
                src_ref=out_com.at[cur], dst_ref=out_com.at[nxt],
                send_sem=o_send.at[cur], recv_sem=o_recv.at[nxt],
                device_id=(right,), device_id_type=pl.DeviceIdType.MESH)
            o_rdma.start()

            @pl.when(t < N_DEV - 1)
            def _():
                x_rdma = pltpu.make_async_remote_copy(
                    src_ref=x_com.at[cur], dst_ref=x_com.at[nxt],
                    send_sem=x_send.at[cur], recv_sem=x_recv.at[nxt],
                    device_id=(right,),
                    device_id_type=pl.DeviceIdType.MESH)
                i_rdma = pltpu.make_async_remote_copy(
                    src_ref=idx_com.at[cur], dst_ref=idx_com.at[nxt],
                    send_sem=i_send.at[cur], recv_sem=i_recv.at[nxt],
                    device_id=(right,),
                    device_id_type=pl.DeviceIdType.MESH)
                x_rdma.start()
                i_rdma.start()
                x_rdma.wait()
                i_rdma.wait()

            o_rdma.wait()

            @pl.when(t <= N_DEV - 2)
            def _():
                pl.semaphore_signal(credit.at[cur], inc=1,
                                    device_id=(left,),
                                    device_id_type=pl.DeviceIdType.MESH)

            @pl.when(t < N_DEV - 1)
            def _():
                out_com[nxt] = contribution(x_com[nxt], idx_com[nxt],
                                            out_com[nxt])
            return carry

        lax.fori_loop(0, N_DEV, step, 0)

        out_ref[...] = out_com[0]

    return pl.pallas_call(
        body,
        out_shape=jax.ShapeDtypeStruct((T, H), jnp.float32),
        in_specs=[
            pl.BlockSpec(memory_space=pltpu.VMEM),
            pl.BlockSpec(memory_space=pltpu.VMEM),
            pl.BlockSpec(memory_space=pltpu.VMEM),
            pl.BlockSpec(memory_space=pltpu.VMEM),
        ],
        out_specs=pl.BlockSpec(memory_space=pltpu.VMEM),
        scratch_shapes=[
            pltpu.VMEM((2, T, D), jnp.float32),
            pltpu.VMEM((2, T, 2), jnp.int32),
            pltpu.VMEM((2, T, H), jnp.float32),
            pltpu.SemaphoreType.DMA((2,)),
            pltpu.SemaphoreType.DMA((2,)),
            pltpu.SemaphoreType.DMA((2,)),
            pltpu.SemaphoreType.DMA((2,)),
            pltpu.SemaphoreType.DMA((2,)),
            pltpu.SemaphoreType.DMA((2,)),
            pltpu.SemaphoreType.REGULAR((2,)),
        ],
        compiler_params=pltpu.CompilerParams(collective_id=0),
    )(x, router_W, route_idx, expert_W)


# baseline (device time: 3051083 ns/iter reference)
import jax
import jax.numpy as jnp
from jax import lax
from jax.experimental import pallas as pl
from jax.experimental.pallas import tpu as pltpu

N_DEV = 16
E_LOC = 8
E_CHUNK = 2
N_ROUND = E_LOC // E_CHUNK
N_STEP = N_ROUND * N_DEV
TM = 256


def kernel(x, router_W, route_idx, expert_W):
    T, D = x.shape
    E = router_W.shape[1]
    H = expert_W.shape[2]

    def body(x_ref, rw_ref, idx_ref, ew_ref, out_ref,
             w_work, w_inbox, gate_ref,
             send_sem, recv_sem, credit):
        me = lax.axis_index("i")
        left = lax.rem(me - 1 + N_DEV, N_DEV)
        right = lax.rem(me + 1, N_DEV)

        bar = pltpu.get_barrier_semaphore()
        pl.semaphore_signal(bar, inc=1, device_id=(left,),
                            device_id_type=pl.DeviceIdType.MESH)
        pl.semaphore_signal(bar, inc=1, device_id=(right,),
                            device_id_type=pl.DeviceIdType.MESH)
        pl.semaphore_wait(bar, 2)

        eids = lax.broadcasted_iota(jnp.int32, (TM, E), 1)

        for r in range(T // TM):
            rows = pl.ds(r * TM, TM)
            xt = x_ref[rows, :]
            scores = jnp.dot(xt, rw_ref[...],
                             preferred_element_type=jnp.float32)
            idx0 = idx_ref[rows, 0:1]
            idx1 = idx_ref[rows, 1:2]
            s0 = jnp.sum(jnp.where(eids == idx0, scores, 0.0),
                         axis=1, keepdims=True)
            s1 = jnp.sum(jnp.where(eids == idx1, scores, 0.0),
                         axis=1, keepdims=True)
            m = jnp.maximum(s0, s1)
            e0 = jnp.exp(s0 - m)
            e1 = jnp.exp(s1 - m)
            w0 = e0 / (e0 + e1)
            gate_ref[rows, :] = (
                w0 * (eids == idx0).astype(jnp.float32)
                + (1.0 - w0) * (eids == idx1).astype(jnp.float32))

        out_ref[...] = jnp.zeros((T, H), jnp.float32)

        def step(g, carry):
            t = lax.rem(g, N_DEV)
            rnd = lax.div(g, N_DEV)
            src = lax.rem(me - t + N_DEV, N_DEV)
            has_rdma = t <= N_DEV - 2

            @pl.when(t == 0)
            def _():
                w_work[...] = ew_ref[pl.ds(rnd * E_CHUNK, E_CHUNK)]

            rdma = pltpu.make_async_remote_copy(
                src_ref=w_work, dst_ref=w_inbox,
                send_sem=send_sem, recv_sem=recv_sem,
                device_id=(right,), device_id_type=pl.DeviceIdType.MESH)

            @pl.when(has_rdma)
            def _():
                @pl.when(g >= 1)
                def _():
                    pl.semaphore_wait(credit, 1)
                rdma.start()

            for r in range(T // TM):
                rows = pl.ds(r * TM, TM)
                xt = x_ref[rows, :]
                gt = gate_ref[rows, :]
                acc = out_ref[rows, :]
                for el in range(E_CHUNK):
                    eg = src * E_LOC + rnd * E_CHUNK + el
                    gv = jnp.sum(jnp.where(eids == eg, gt, 0.0),
                                 axis=1, keepdims=True)
                    y = jnp.dot(xt, w_work[el],
                                preferred_element_type=jnp.float32)
                    acc = acc + gv * y
                out_ref[rows, :] = acc

            @pl.when(has_rdma)
            def _():
                rdma.wait()
                w_work[...] = w_inbox[...]
                @pl.when(g <= N_STEP - 3)
                def _():
                    pl.semaphore_signal(credit, inc=1,
                                        device_id=(left,),
                                        device_id_type=pl.DeviceIdType.MESH)
            return carry

        lax.fori_loop(0, N_STEP, step, 0)

    return pl.pallas_call(
        body,
        out_shape=jax.ShapeDtypeStruct((T, H), jnp.float32),
        in_specs=[
            pl.BlockSpec(memory_space=pltpu.VMEM),
            pl.BlockSpec(memory_space=pltpu.VMEM),
            pl.BlockSpec(memory_space=pltpu.VMEM),
            pl.BlockSpec(memory_space=pltpu.VMEM),
        ],
        out_specs=pl.BlockSpec(memory_space=pltpu.VMEM),
        scratch_shapes=[
            pltpu.VMEM((E_CHUNK, D, H), jnp.float32),
            pltpu.VMEM((E_CHUNK, D, H), jnp.float32),
            pltpu.VMEM((T, E), jnp.float32),
            pltpu.SemaphoreType.DMA,
            pltpu.SemaphoreType.DMA,
            pltpu.SemaphoreType.REGULAR,
        ],
        compiler_params=pltpu.CompilerParams(collective_id=0),
    )(x, router_W, route_idx, expert_W)


# device time: 1547136 ns/iter; 1.9721x vs baseline; 1.9721x over previous
import jax
import jax.numpy as jnp
from jax import lax
from jax.experimental import pallas as pl
from jax.experimental.pallas import tpu as pltpu

N_DEV = 16
E_LOC = 8
E_CHUNK = 4
N_ROUND = E_LOC // E_CHUNK
N_STEP = N_ROUND * N_DEV
TM = 256


def kernel(x, router_W, route_idx, expert_W):
    T, D = x.shape
    E = router_W.shape[1]
    H = expert_W.shape[2]

    def body(x_ref, rw_ref, idx_ref, ew_ref, out_ref,
             w_work, w_inbox, gate_ref, x_bf,
             send_sem, recv_sem, credit):
        me = lax.axis_index("i")
        left = lax.rem(me - 1 + N_DEV, N_DEV)
        right = lax.rem(me + 1, N_DEV)

        bar = pltpu.get_barrier_semaphore()
        pl.semaphore_signal(bar, inc=1, device_id=(left,),
                            device_id_type=pl.DeviceIdType.MESH)
        pl.semaphore_signal(bar, inc=1, device_id=(right,),
                            device_id_type=pl.DeviceIdType.MESH)
        pl.semaphore_wait(bar, 2)

        eids = lax.broadcasted_iota(jnp.int32, (TM, E), 1)

        for r in range(T // TM):
            rows = pl.ds(r * TM, TM)
            xt = x_ref[rows, :]
            scores = jnp.dot(xt, rw_ref[...],
                             preferred_element_type=jnp.float32)
            idx0 = idx_ref[rows, 0:1]
            idx1 = idx_ref[rows, 1:2]
            s0 = jnp.sum(jnp.where(eids == idx0, scores, 0.0),
                         axis=1, keepdims=True)
            s1 = jnp.sum(jnp.where(eids == idx1, scores, 0.0),
                         axis=1, keepdims=True)
            m = jnp.maximum(s0, s1)
            e0 = jnp.exp(s0 - m)
            e1 = jnp.exp(s1 - m)
            w0 = e0 / (e0 + e1)
            gate_ref[rows, :] = (
                w0 * (eids == idx0).astype(jnp.float32)
                + (1.0 - w0) * (eids == idx1).astype(jnp.float32))

        out_ref[...] = jnp.zeros((T, H), jnp.float32)
        x_bf[...] = x_ref[...].astype(jnp.bfloat16)

        def step(g, carry):
            t = lax.rem(g, N_DEV)
            rnd = lax.div(g, N_DEV)
            src = lax.rem(me - t + N_DEV, N_DEV)
            has_rdma = t <= N_DEV - 2

            @pl.when(t == 0)
            def _():
                w_work[...] = ew_ref[pl.ds(rnd * E_CHUNK, E_CHUNK)].astype(
                    jnp.bfloat16)

            rdma = pltpu.make_async_remote_copy(
                src_ref=w_work, dst_ref=w_inbox,
                send_sem=send_sem, recv_sem=recv_sem,
                device_id=(right,), device_id_type=pl.DeviceIdType.MESH)

            @pl.when(has_rdma)
            def _():
                @pl.when(g >= 1)
                def _():
                    pl.semaphore_wait(credit, 1)
                rdma.start()

            for r in range(T // TM):
                rows = pl.ds(r * TM, TM)
                xt = x_bf[rows, :]
                gt = gate_ref[rows, :]
                acc = out_ref[rows, :]
                for el in range(E_CHUNK):
                    eg = src * E_LOC + rnd * E_CHUNK + el
                    gv = jnp.sum(jnp.where(eids == eg, gt, 0.0),
                                 axis=1, keepdims=True)
                    y = jnp.dot(xt, w_work[el],
                                preferred_element_type=jnp.float32)
                    acc = acc + gv * y
                out_ref[rows, :] = acc

            @pl.when(has_rdma)
            def _():
                rdma.wait()
                w_work[...] = w_inbox[...]
                @pl.when(g <= N_STEP - 3)
                def _():
                    pl.semaphore_signal(credit, inc=1,
                                        device_id=(left,),
                                        device_id_type=pl.DeviceIdType.MESH)
            return carry

        lax.fori_loop(0, N_STEP, step, 0)

    return pl.pallas_call(
        body,
        out_shape=jax.ShapeDtypeStruct((T, H), jnp.float32),
        in_specs=[
            pl.BlockSpec(memory_space=pltpu.VMEM),
            pl.BlockSpec(memory_space=pltpu.VMEM),
            pl.BlockSpec(memory_space=pltpu.VMEM),
            pl.BlockSpec(memory_space=pltpu.VMEM),
        ],
        out_specs=pl.BlockSpec(memory_space=pltpu.VMEM),
        scratch_shapes=[
            pltpu.VMEM((E_CHUNK, D, H), jnp.bfloat16),
            pltpu.VMEM((E_CHUNK, D, H), jnp.bfloat16),
            pltpu.VMEM((T, E), jnp.float32),
            pltpu.VMEM((T, D), jnp.bfloat16),
            pltpu.SemaphoreType.DMA,
            pltpu.SemaphoreType.DMA,
            pltpu.SemaphoreType.REGULAR,
        ],
        compiler_params=pltpu.CompilerParams(collective_id=0),
    )(x, router_W, route_idx, expert_W)
